# baseline (speedup 1.0000x reference)
import functools

import numpy as np
import jax
import jax.numpy as jnp
from jax import lax
from jax.experimental import pallas as pl
from jax.experimental.pallas import tpu as pltpu
from jax.experimental.pallas import tpu_sc as plsc

N = 10000
E = 320000
D = 128
H = 8
HD = 16
HOP = 4
ALPHA = 0.15
NP = 10112
NG = 10240
NC = 2
NS = 16
NW = NC * NS
RPS = NP // NS

RW = 10240

_mesh = plsc.VectorSubcoreMesh(core_axis_name="c", subcore_axis_name="s")

_S128_np = np.zeros((D, D), np.float32)
for _a in range(D):
    for _b in range(D):
        if _a // HD == _b // HD:
            _S128_np[_a, _b] = 1.0


def _wid():
    return lax.axis_index("s") * NC + lax.axis_index("c")


def _nsup(w, sup):
    return jnp.where(w == NW - 1, (RW // 4) // sup, RW // sup)


@functools.cache
def _sc_gather_fn(T, Dv, B, chunk):
    rpw = B // NW
    nch = rpw // chunk

    @functools.partial(
        pl.kernel,
        out_type=jax.ShapeDtypeStruct((B, Dv), jnp.float32),
        mesh=_mesh,
        scratch_types=[
            pltpu.VMEM((chunk,), jnp.int32),
            pltpu.VMEM((chunk, Dv), jnp.float32),
            pltpu.SemaphoreType.DMA,
        ],
    )
    def k(table_h, idx_h, out_h, idx_v, rows_v, sem):
        base = _wid() * rpw

        def body(i, c):
            off = pl.multiple_of(base + i * chunk, 8)
            pltpu.sync_copy(idx_h.at[pl.ds(off, chunk)], idx_v)
            pltpu.async_copy(table_h.at[idx_v], rows_v, sem).wait()
            pltpu.sync_copy(rows_v, out_h.at[pl.ds(off, chunk)])
            return c

        lax.fori_loop(0, nch, body, 0)

    return k


def _sc_gather(table, idx, chunk):
    return _sc_gather_fn(table.shape[0], table.shape[1], idx.shape[0], chunk)(
        table, idx)


_SUP = 256


@functools.cache
def _sc_gather_big_fn(T, Dv):
    @functools.partial(
        pl.kernel,
        out_type=jax.ShapeDtypeStruct((E, Dv), jnp.float32),
        mesh=_mesh,
        scratch_types=[
            pltpu.VMEM((2, _SUP), jnp.int32),
            pltpu.VMEM((2, _SUP, Dv), jnp.float32),
            pltpu.SemaphoreType.DMA((2,)),
            pltpu.SemaphoreType.DMA((2,)),
            pltpu.SemaphoreType.DMA((2,)),
        ],
    )
    def k(table_h, idx_h, out_h, idx_v, rows_v, isem, gsem, wsem):
        w = _wid()
        base = w * RW
        nsup = _nsup(w, _SUP)

        def off(i):
            return pl.multiple_of(base + i * _SUP, _SUP)

        def idx_load(i, p):
            pltpu.async_copy(idx_h.at[pl.ds(off(i), _SUP)], idx_v.at[p],
                             isem.at[p])

        idx_load(0, 0)

        def body(i, c):
            p = lax.rem(i, 2)

            @pl.when(i + 1 < nsup)
            def _():
                idx_load(i + 1, 1 - p)

            pltpu.make_async_copy(idx_h.at[pl.ds(off(i), _SUP)], idx_v.at[p],
                                  isem.at[p]).wait()

            @pl.when(i >= 2)
            def _():
                pltpu.make_async_copy(
                    rows_v.at[p], out_h.at[pl.ds(off(i - 2), _SUP)],
                    wsem.at[p]).wait()

            for j in range(_SUP // 128):
                pltpu.async_copy(
                    table_h.at[idx_v.at[p, pl.ds(j * 128, 128)]],
                    rows_v.at[p, pl.ds(j * 128, 128)], gsem.at[p])
            for j in range(_SUP // 128):
                pltpu.make_async_copy(
                    table_h.at[idx_v.at[p, pl.ds(j * 128, 128)]],
                    rows_v.at[p, pl.ds(j * 128, 128)], gsem.at[p]).wait()
            pltpu.async_copy(rows_v.at[p], out_h.at[pl.ds(off(i), _SUP)],
                             wsem.at[p])
            return c

        lax.fori_loop(0, nsup, body, 0)

        def drain(t, c):
            i = nsup - 2 + t

            @pl.when(i >= 0)
            def _():
                p = lax.rem(i, 2)
                pltpu.make_async_copy(
                    rows_v.at[p], out_h.at[pl.ds(off(i), _SUP)],
                    wsem.at[p]).wait()
            return c

        lax.fori_loop(0, 2, drain, 0)

    return k


def _sc_gather_big(table, idx):
    return _sc_gather_big_fn(table.shape[0], table.shape[1])(table, idx)


@functools.cache
def _sc_score_fn(with_rel):
    scratch = [
        pltpu.VMEM((2, 128), jnp.int32),
        pltpu.VMEM((2, 128), jnp.int32),
        pltpu.VMEM((2, 128, D), jnp.float32),
        pltpu.VMEM((2, 128, D), jnp.float32),
        pltpu.SemaphoreType.DMA((2,)),
        pltpu.SemaphoreType.DMA((2,)),
        pltpu.SemaphoreType.DMA((2,)),
    ]
    if with_rel:
        scratch.insert(4, pltpu.VMEM((2, 128, D), jnp.float32))

    @functools.partial(
        pl.kernel,
        out_type=jax.ShapeDtypeStruct((E, D), jnp.float32),
        mesh=_mesh,
        scratch_types=scratch,
    )
    def k(*args):
        if with_rel:
            (k_h, q_h, src_h, dst_h, re_h, ex_h,
             sidx, didx, krows, qrows, rerows, isem, gsem, wsem) = args
        else:
            (k_h, q_h, src_h, dst_h, ex_h,
             sidx, didx, krows, qrows, isem, gsem, wsem) = args
            rerows = None
        w = _wid()
        base = w * RW
        nsup = _nsup(w, 128)

        def eoff(i):
            return pl.multiple_of(base + i * 128, 128)

        def idx_load(i, p):
            pltpu.async_copy(src_h.at[pl.ds(eoff(i), 128)], sidx.at[p],
                             isem.at[p])
            pltpu.async_copy(dst_h.at[pl.ds(eoff(i), 128)], didx.at[p],
                             isem.at[p])

        def wait_idx(i, p):
            pltpu.make_async_copy(src_h.at[pl.ds(base, 128)], sidx.at[p],
                                  isem.at[p]).wait()
            pltpu.make_async_copy(dst_h.at[pl.ds(base, 128)], didx.at[p],
                                  isem.at[p]).wait()

        def gathers(i, p):
            pltpu.async_copy(k_h.at[sidx.at[p]], krows.at[p], gsem.at[p])
            pltpu.async_copy(q_h.at[didx.at[p]], qrows.at[p], gsem.at[p])
            if with_rel:
                pltpu.async_copy(re_h.at[pl.ds(eoff(i), 128)], rerows.at[p],
                                 gsem.at[p])

        def wait_gathers(i, p):
            pltpu.make_async_copy(k_h.at[sidx.at[p]], krows.at[p],
                                  gsem.at[p]).wait()
            pltpu.make_async_copy(q_h.at[didx.at[p]], qrows.at[p],
                                  gsem.at[p]).wait()
            if with_rel:
                pltpu.make_async_copy(re_h.at[pl.ds(base, 128)], rerows.at[p],
                                      gsem.at[p]).wait()

        idx_load(0, 0)

        def body(i, c):
            p = lax.rem(i, 2)
            wait_idx(i, p)

            @pl.when(i >= 2)
            def _():
                pltpu.make_async_copy(
                    krows.at[p], ex_h.at[pl.ds(eoff(i - 2), 128)],
                    wsem.at[p]).wait()
            gathers(i, p)

            @pl.when(i + 1 < nsup)
            def _():
                idx_load(i + 1, 1 - p)
            wait_gathers(i, p)

            def score(g, c2):
                rb = pl.multiple_of(g * 8, 8)
                for rs in range(8):
                    r = rb + rs
                    for j in range(8):
                        sl = pl.ds(16 * j, 16)
                        kv = krows[p, r, sl]
                        if with_rel:
                            kv = kv + rerows[p, r, sl]
                        krows[p, r, sl] = kv * qrows[p, r, sl]
                return c2

            lax.fori_loop(0, 16, score, 0)
            pltpu.async_copy(krows.at[p], ex_h.at[pl.ds(eoff(i), 128)],
                             wsem.at[p])
            return c

        lax.fori_loop(0, nsup, body, 0)

        def drain(t, c):
            i = nsup - 2 + t
            p = lax.rem(i, 2)
            pltpu.make_async_copy(krows.at[p],
                                  ex_h.at[pl.ds(eoff(i), 128)],
                                  wsem.at[p]).wait()
            return c

        lax.fori_loop(0, 2, drain, 0)

    return k


def _sc_score(kq, q, src, dst, re):
    if re is None:
        return _sc_score_fn(False)(kq, q, src, dst)
    return _sc_score_fn(True)(kq, q, src, dst, re)


def _zero_rows(buf, nrows, Dv):
    zv = jnp.zeros((16,), jnp.float32)
    for r in range(nrows):
        for j in range(Dv // 16):
            buf[r, pl.ds(j * 16, 16)] = zv


@functools.cache
def _sc_scatter_add_fn(B, Dv):
    CH = 80
    rpw = B // NW
    nch = rpw // CH

    @functools.partial(
        pl.kernel,
        out_type=jax.ShapeDtypeStruct((2 * NP, Dv), jnp.float32),
        mesh=_mesh,
        scratch_types=[
            pltpu.VMEM((CH,), jnp.int32),
            pltpu.VMEM((CH, Dv), jnp.float32),
            pltpu.VMEM_SHARED((NP, Dv), jnp.float32),
            pltpu.SemaphoreType.DMA,
        ],
    )
    def k(vals_h, idx_h, out_h, idx_v, vals_v, acc, sem):
        cid = lax.axis_index("c")
        sid = lax.axis_index("s")
        _zero_rows(vals_v, CH, Dv)
        for i, (o, ln) in enumerate([(0, 80), (80, 80), (160, 80), (240, 80),
                                     (320, 80), (400, 80), (480, 80),
                                     (560, 72)]):
            pltpu.sync_copy(vals_v.at[pl.ds(0, ln)],
                            acc.at[pl.ds(sid * RPS + o, ln)])
        plsc.subcore_barrier()

        base = _wid() * rpw

        def body(i, c):
            off = pl.multiple_of(base + i * CH, 8)
            pltpu.sync_copy(idx_h.at[pl.ds(off, CH)], idx_v)
            pltpu.sync_copy(vals_h.at[pl.ds(off, CH)], vals_v)
            pltpu.sync_copy(vals_v, acc.at[idx_v], add=True)
            return c

        lax.fori_loop(0, nch, body, 0)
        plsc.subcore_barrier()
        pltpu.sync_copy(acc.at[pl.ds(sid * RPS, RPS)],
                        out_h.at[pl.ds(cid * NP + sid * RPS, RPS)])

    return k


def _sc_scatter_add(vals, idx):
    return _sc_scatter_add_fn(vals.shape[0], vals.shape[1])(vals, idx)


@functools.cache
def _sc_hop_fused_fn():
    @functools.partial(
        pl.kernel,
        out_type=jax.ShapeDtypeStruct((2 * NP, D), jnp.float32),
        mesh=_mesh,
        scratch_types=[
            pltpu.VMEM((2, 128), jnp.int32),
            pltpu.VMEM((8, 128), jnp.int32),
            pltpu.VMEM((128, D), jnp.float32),
            pltpu.VMEM((128, D), jnp.float32),
            pltpu.VMEM_SHARED((NP, D), jnp.float32),
            pltpu.SemaphoreType.DMA((2,)),
            pltpu.SemaphoreType.DMA,
            pltpu.SemaphoreType.DMA,
            pltpu.SemaphoreType.DMA,
            pltpu.SemaphoreType.DMA,
        ],
    )
    def k(feat_h, att_h, src_h, dst2_h, out_h,
          sidx, didx, attv, fev, acc,
          ssem, dsem, asem, gsem, wsem):
        cid = lax.axis_index("c")
        sid = lax.axis_index("s")
        w = _wid()
        base = w * RW
        rbase = w * (RW // 128)
        nsup = _nsup(w, 128)

        _zero_rows(fev, 128, D)
        for o, ln in [(0, 128), (128, 128), (256, 128), (384, 128),
                      (512, 120)]:
            pltpu.sync_copy(fev.at[pl.ds(0, ln)],
                            acc.at[pl.ds(sid * RPS + o, ln)])
        plsc.subcore_barrier()

        def eoff(i):
            return pl.multiple_of(base + i * 128, 128)

        def load_sidx(i):
            pltpu.async_copy(src_h.at[pl.ds(eoff(i), 128)],
                             sidx.at[lax.rem(i, 2)], ssem.at[lax.rem(i, 2)])

        def load_dblock(i):
            pltpu.async_copy(
                dst2_h.at[pl.ds(pl.multiple_of(rbase + i, 8), 8)], didx, dsem)

        def load_att(i):
            pltpu.async_copy(att_h.at[pl.ds(eoff(i), 128)], attv, asem)

        def wait_sidx(i):
            pltpu.make_async_copy(src_h.at[pl.ds(base, 128)],
                                  sidx.at[lax.rem(i, 2)],
                                  ssem.at[lax.rem(i, 2)]).wait()

        def wait_scatter():
            pltpu.make_async_copy(fev, acc.at[didx.at[0]], wsem).wait()

        load_sidx(0)
        load_sidx(1)
        load_dblock(0)
        load_att(0)

        def body(i, c):
            p = lax.rem(i, 2)

            @pl.when(i >= 1)
            def _():
                wait_scatter()
            wait_sidx(i)
            pltpu.async_copy(feat_h.at[sidx.at[p]], fev, gsem)

            @pl.when(jnp.logical_and(lax.rem(i, 8) == 0, i > 0))
            def _():
                load_dblock(i)

            pltpu.make_async_copy(feat_h.at[sidx.at[p]], fev, gsem).wait()

            @pl.when(i + 2 < nsup)
            def _():
                load_sidx(i + 2)
            pltpu.make_async_copy(att_h.at[pl.ds(base, 128)], attv,
                                  asem).wait()

            def mul(g, c2):
                rb = pl.multiple_of(g * 8, 8)
                for rs in range(8):
                    r = rb + rs
                    for j in range(8):
                        sl = pl.ds(16 * j, 16)
                        fev[r, sl] = fev[r, sl] * attv[r, sl]
                return c2

            lax.fori_loop(0, 16, mul, 0)

            @pl.when(i + 1 < nsup)
            def _():
                load_att(i + 1)

            @pl.when(lax.rem(i, 8) == 0)
            def _():
                pltpu.make_async_copy(dst2_h.at[pl.ds(rbase, 8)], didx,
                                      dsem).wait()

            pltpu.async_copy(fev, acc.at[didx.at[lax.rem(i, 8)]], wsem,
                             add=True)
            return c

        lax.fori_loop(0, nsup, body, 0)
        wait_scatter()
        plsc.subcore_barrier()
        pltpu.sync_copy(acc.at[pl.ds(sid * RPS, RPS)],
                        out_h.at[pl.ds(cid * NP + sid * RPS, RPS)])

    return k


def _sc_hop_fused(feat, att, src, dst2):
    return _sc_hop_fused_fn()(feat, att, src, dst2)


def _tc_qkv(h, Wt, Wh, Wv):
    BN = 2528

    def body(h_r, wt_r, wh_r, wv_r, q_r, k_r, v_r):
        hb = h_r[...]
        q_r[...] = jnp.dot(hb, wt_r[...],
                           preferred_element_type=jnp.float32) * 0.25
        k_r[...] = jnp.dot(hb, wh_r[...], preferred_element_type=jnp.float32)
        v_r[...] = jnp.dot(hb, wv_r[...], preferred_element_type=jnp.float32)

    bs_h = pl.BlockSpec((BN, D), lambda i: (i, 0))
    bs_w = pl.BlockSpec((D, D), lambda i: (0, 0))
    return pl.pallas_call(
        body, grid=(NP // BN,),
        in_specs=[bs_h, bs_w, bs_w, bs_w],
        out_specs=[bs_h, bs_h, bs_h],
        out_shape=[jax.ShapeDtypeStruct((NP, D), jnp.float32)] * 3,
    )(h, Wt, Wh, Wv)


def _tc_relproj(rel_emb, Wr):
    def body(re_r, wr_r, o_r):
        o_r[...] = jnp.dot(re_r[...], wr_r[...],
                           preferred_element_type=jnp.float32)

    return pl.pallas_call(
        body,
        out_shape=jax.ShapeDtypeStruct((16, D), jnp.float32),
    )(rel_emb, Wr)


def _tc_tile_r(R):
    def body(r_r, o_r):
        o_r[...] = r_r[...]

    bs = pl.BlockSpec((16, D), lambda i: (0, 0))
    bo = pl.BlockSpec((16, D), lambda i: (i, 0))
    return pl.pallas_call(
        body, grid=(128,), in_specs=[bs], out_specs=bo,
        out_shape=jax.ShapeDtypeStruct((2048, D), jnp.float32),
    )(R)


def _tc_rel2(rel2d):
    BE = 2500

    def body(r_r, o_r):
        o_r[...] = r_r[...] + 16 * jax.lax.broadcasted_iota(
            jnp.int32, (BE, 128), 1)

    bs = pl.BlockSpec((BE, 128), lambda i: (i, 0))
    return pl.pallas_call(
        body, grid=(2500 // BE,), in_specs=[bs], out_specs=bs,
        out_shape=jax.ShapeDtypeStruct((2500, 128), jnp.int32),
    )(rel2d)


def _tc_exps(kq, S):
    BE = 4000
    bs_e = pl.BlockSpec((BE, D), lambda i: (i, 0))
    bs_s = pl.BlockSpec((D, D), lambda i: (0, 0))

    def body(kq_r, s_r, ex_r):
        ex_r[...] = jnp.exp(
            jnp.dot(kq_r[...], s_r[...], preferred_element_type=jnp.float32))

    return pl.pallas_call(
        body, grid=(E // BE,), in_specs=[bs_e, bs_s], out_specs=bs_e,
        out_shape=jax.ShapeDtypeStruct((E, D), jnp.float32),
    )(kq, S)


def _tc_sddmm(ke, qe, re, S):
    BE = 4000
    bs_e = pl.BlockSpec((BE, D), lambda i: (i, 0))
    bs_s = pl.BlockSpec((D, D), lambda i: (0, 0))
    if re is None:
        def body(ke_r, qe_r, s_r, ex_r):
            t = ke_r[...] * qe_r[...]
            ex_r[...] = jnp.exp(
                jnp.dot(t, s_r[...], preferred_element_type=jnp.float32))

        ins, specs = (ke, qe, S), [bs_e, bs_e, bs_s]
    else:
        def body(ke_r, re_r, qe_r, s_r, ex_r):
            t = (ke_r[...] + re_r[...]) * qe_r[...]
            ex_r[...] = jnp.exp(
                jnp.dot(t, s_r[...], preferred_element_type=jnp.float32))

        ins, specs = (ke, re, qe, S), [bs_e, bs_e, bs_e, bs_s]
    return pl.pallas_call(
        body, grid=(E // BE,), in_specs=specs, out_specs=bs_e,
        out_shape=jax.ShapeDtypeStruct((E, D), jnp.float32),
    )(*ins)


def _tc_add(a, b):
    BN = 2528

    def body(a_r, b_r, o_r):
        o_r[...] = a_r[...] + b_r[...]

    bs = pl.BlockSpec((BN, a.shape[1]), lambda i: (i, 0))
    return pl.pallas_call(
        body, grid=(a.shape[0] // BN,), in_specs=[bs, bs], out_specs=bs,
        out_shape=jax.ShapeDtypeStruct(a.shape, jnp.float32),
    )(a, b)


def _tc_att(ex, den_e):
    BE = 4000

    def body(ex_r, de_r, o_r):
        o_r[...] = ex_r[...] / (de_r[...] + 1e-9)

    bs = pl.BlockSpec((BE, D), lambda i: (i, 0))
    return pl.pallas_call(
        body, grid=(E // BE,), in_specs=[bs, bs], out_specs=bs,
        out_shape=jax.ShapeDtypeStruct((E, D), jnp.float32),
    )(ex, den_e)


def _tc_combine(p0, p1, feat0):
    BN = 2528

    def body(p0_r, p1_r, f0_r, o_r):
        o_r[...] = ALPHA * f0_r[...] + (1.0 - ALPHA) * (p0_r[...] + p1_r[...])

    bs = pl.BlockSpec((BN, D), lambda i: (i, 0))
    return pl.pallas_call(
        body, grid=(NP // BN,), in_specs=[bs, bs, bs], out_specs=bs,
        out_shape=jax.ShapeDtypeStruct((NP, D), jnp.float32),
    )(p0, p1, feat0)


def _tc_layerout(p0, p1, feat0, h):
    BN = 2528

    def body(p0_r, p1_r, f0_r, h_r, o_r):
        f = ALPHA * f0_r[...] + (1.0 - ALPHA) * (p0_r[...] + p1_r[...])
        o_r[...] = jnp.where(f > 0, f, jnp.exp(f) - 1.0) + h_r[...]

    bs = pl.BlockSpec((BN, D), lambda i: (i, 0))
    return pl.pallas_call(
        body, grid=(NP // BN,), in_specs=[bs, bs, bs, bs], out_specs=bs,
        out_shape=jax.ShapeDtypeStruct((NP, D), jnp.float32),
    )(p0, p1, feat0, h)


def kernel(ent_ids, rel_ids, edge_index, batch_cls, node_emb, rel_emb, params):
    S = jnp.asarray(_S128_np)
    src = edge_index[0]
    dst = edge_index[1]
    dst2 = jnp.reshape(dst, (E // 128, 128))
    ent_p = jnp.concatenate(
        [ent_ids, jnp.zeros((NG - N,), ent_ids.dtype)])

    h = _sc_gather(node_emb, ent_p, 80)[:NP]
    R = _tc_relproj(rel_emb, params['l0_Wr'])
    rel2 = jnp.reshape(
        _tc_rel2(jnp.reshape(rel_ids, (E // 128, 128))), (E,))
    re_full = _sc_gather_big(_tc_tile_r(R), rel2)

    for l in range(3):
        p = params
        q, k, v = _tc_qkv(h, p[f'l{l}_Wt'], p[f'l{l}_Wh'], p[f'l{l}_Wv'])
        kq = _sc_score(k, q, src, dst, re_full if l == 0 else None)
        ex = _tc_exps(kq, S)
        dpart = _sc_scatter_add(ex, dst)
        denom = _tc_add(dpart[:NP], dpart[NP:])
        den_e = _sc_gather_big(denom, dst)
        att = _tc_att(ex, den_e)
        feat = v
        for hop in range(HOP):
            apart = _sc_hop_fused(feat, att, src, dst2)
            if hop < HOP - 1:
                feat = _tc_combine(apart[:NP], apart[NP:], v)
            else:
                h = _tc_layerout(apart[:NP], apart[NP:], v, h)

    return _sc_gather(h, batch_cls, 8)


# speedup vs baseline: 1.0636x; 1.0636x over previous
import functools

import numpy as np
import jax
import jax.numpy as jnp
from jax import lax
from jax.experimental import pallas as pl
from jax.experimental.pallas import tpu as pltpu
from jax.experimental.pallas import tpu_sc as plsc

N = 10000
E = 320000
D = 128
H = 8
HD = 16
HOP = 4
ALPHA = 0.15
NP = 10112
NG = 10240
NC = 2
NS = 16
NW = NC * NS
RPS = NP // NS

RW = 10240

_mesh = plsc.VectorSubcoreMesh(core_axis_name="c", subcore_axis_name="s")

_S128_np = np.zeros((D, D), np.float32)
for _a in range(D):
    for _b in range(D):
        if _a // HD == _b // HD:
            _S128_np[_a, _b] = 1.0


def _wid():
    return lax.axis_index("s") * NC + lax.axis_index("c")


def _nsup(w, sup):
    return jnp.where(w == NW - 1, (RW // 4) // sup, RW // sup)


@functools.cache
def _sc_gather_fn(T, Dv, B, chunk):
    rpw = B // NW
    nch = rpw // chunk

    @functools.partial(
        pl.kernel,
        out_type=jax.ShapeDtypeStruct((B, Dv), jnp.float32),
        mesh=_mesh,
        scratch_types=[
            pltpu.VMEM((chunk,), jnp.int32),
            pltpu.VMEM((chunk, Dv), jnp.float32),
            pltpu.SemaphoreType.DMA,
        ],
    )
    def k(table_h, idx_h, out_h, idx_v, rows_v, sem):
        base = _wid() * rpw

        def body(i, c):
            off = pl.multiple_of(base + i * chunk, 8)
            pltpu.sync_copy(idx_h.at[pl.ds(off, chunk)], idx_v)
            pltpu.async_copy(table_h.at[idx_v], rows_v, sem).wait()
            pltpu.sync_copy(rows_v, out_h.at[pl.ds(off, chunk)])
            return c

        lax.fori_loop(0, nch, body, 0)

    return k


def _sc_gather(table, idx, chunk):
    return _sc_gather_fn(table.shape[0], table.shape[1], idx.shape[0], chunk)(
        table, idx)


_SUP = 256


@functools.cache
def _sc_gather_big_fn(T, Dv):
    @functools.partial(
        pl.kernel,
        out_type=jax.ShapeDtypeStruct((E, Dv), jnp.float32),
        mesh=_mesh,
        scratch_types=[
            pltpu.VMEM((2, _SUP), jnp.int32),
            pltpu.VMEM((2, _SUP, Dv), jnp.float32),
            pltpu.SemaphoreType.DMA((2,)),
            pltpu.SemaphoreType.DMA((2,)),
            pltpu.SemaphoreType.DMA((2,)),
        ],
    )
    def k(table_h, idx_h, out_h, idx_v, rows_v, isem, gsem, wsem):
        w = _wid()
        base = w * RW
        nsup = _nsup(w, _SUP)

        def off(i):
            return pl.multiple_of(base + i * _SUP, _SUP)

        def idx_load(i, p):
            pltpu.async_copy(idx_h.at[pl.ds(off(i), _SUP)], idx_v.at[p],
                             isem.at[p])

        idx_load(0, 0)

        def body(i, c):
            p = lax.rem(i, 2)

            @pl.when(i + 1 < nsup)
            def _():
                idx_load(i + 1, 1 - p)

            pltpu.make_async_copy(idx_h.at[pl.ds(off(i), _SUP)], idx_v.at[p],
                                  isem.at[p]).wait()

            @pl.when(i >= 2)
            def _():
                pltpu.make_async_copy(
                    rows_v.at[p], out_h.at[pl.ds(off(i - 2), _SUP)],
                    wsem.at[p]).wait()

            for j in range(_SUP // 128):
                pltpu.async_copy(
                    table_h.at[idx_v.at[p, pl.ds(j * 128, 128)]],
                    rows_v.at[p, pl.ds(j * 128, 128)], gsem.at[p])
            for j in range(_SUP // 128):
                pltpu.make_async_copy(
                    table_h.at[idx_v.at[p, pl.ds(j * 128, 128)]],
                    rows_v.at[p, pl.ds(j * 128, 128)], gsem.at[p]).wait()
            pltpu.async_copy(rows_v.at[p], out_h.at[pl.ds(off(i), _SUP)],
                             wsem.at[p])
            return c

        lax.fori_loop(0, nsup, body, 0)

        def drain(t, c):
            i = nsup - 2 + t

            @pl.when(i >= 0)
            def _():
                p = lax.rem(i, 2)
                pltpu.make_async_copy(
                    rows_v.at[p], out_h.at[pl.ds(off(i), _SUP)],
                    wsem.at[p]).wait()
            return c

        lax.fori_loop(0, 2, drain, 0)

    return k


def _sc_gather_big(table, idx):
    return _sc_gather_big_fn(table.shape[0], table.shape[1])(table, idx)


@functools.cache
def _sc_score_fn(with_rel):
    scratch = [
        pltpu.VMEM((2, 128), jnp.int32),
        pltpu.VMEM((2, 128), jnp.int32),
        pltpu.VMEM((2, 128, D), jnp.float32),
        pltpu.VMEM((2, 128, D), jnp.float32),
        pltpu.SemaphoreType.DMA((2,)),
        pltpu.SemaphoreType.DMA((2,)),
        pltpu.SemaphoreType.DMA((2,)),
    ]
    if with_rel:
        scratch.insert(4, pltpu.VMEM((2, 128, D), jnp.float32))

    @functools.partial(
        pl.kernel,
        out_type=jax.ShapeDtypeStruct((E, D), jnp.float32),
        mesh=_mesh,
        scratch_types=scratch,
    )
    def k(*args):
        if with_rel:
            (k_h, q_h, src_h, dst_h, re_h, ex_h,
             sidx, didx, krows, qrows, rerows, isem, gsem, wsem) = args
        else:
            (k_h, q_h, src_h, dst_h, ex_h,
             sidx, didx, krows, qrows, isem, gsem, wsem) = args
            rerows = None
        w = _wid()
        base = w * RW
        nsup = _nsup(w, 128)

        def eoff(i):
            return pl.multiple_of(base + i * 128, 128)

        def idx_load(i, p):
            pltpu.async_copy(src_h.at[pl.ds(eoff(i), 128)], sidx.at[p],
                             isem.at[p])
            pltpu.async_copy(dst_h.at[pl.ds(eoff(i), 128)], didx.at[p],
                             isem.at[p])

        def wait_idx(i, p):
            pltpu.make_async_copy(src_h.at[pl.ds(base, 128)], sidx.at[p],
                                  isem.at[p]).wait()
            pltpu.make_async_copy(dst_h.at[pl.ds(base, 128)], didx.at[p],
                                  isem.at[p]).wait()

        def gathers(i, p):
            pltpu.async_copy(k_h.at[sidx.at[p]], krows.at[p], gsem.at[p])
            pltpu.async_copy(q_h.at[didx.at[p]], qrows.at[p], gsem.at[p])
            if with_rel:
                pltpu.async_copy(re_h.at[pl.ds(eoff(i), 128)], rerows.at[p],
                                 gsem.at[p])

        def wait_gathers(i, p):
            pltpu.make_async_copy(k_h.at[sidx.at[p]], krows.at[p],
                                  gsem.at[p]).wait()
            pltpu.make_async_copy(q_h.at[didx.at[p]], qrows.at[p],
                                  gsem.at[p]).wait()
            if with_rel:
                pltpu.make_async_copy(re_h.at[pl.ds(base, 128)], rerows.at[p],
                                      gsem.at[p]).wait()

        idx_load(0, 0)
        wait_idx(0, 0)
        gathers(0, 0)
        idx_load(1, 1)

        def body(i, c):
            p = lax.rem(i, 2)

            @pl.when(i + 1 < nsup)
            def _():
                wait_idx(i + 1, 1 - p)

                @pl.when(i >= 1)
                def _():
                    pltpu.make_async_copy(
                        krows.at[1 - p], ex_h.at[pl.ds(eoff(i - 1), 128)],
                        wsem.at[1 - p]).wait()
                gathers(i + 1, 1 - p)

            wait_gathers(i, p)

            @pl.when(i + 2 < nsup)
            def _():
                idx_load(i + 2, p)

            def score(g, c2):
                rb = pl.multiple_of(g * 8, 8)
                for rs in range(8):
                    r = rb + rs
                    for j in range(8):
                        sl = pl.ds(16 * j, 16)
                        kv = krows[p, r, sl]
                        if with_rel:
                            kv = kv + rerows[p, r, sl]
                        krows[p, r, sl] = kv * qrows[p, r, sl]
                return c2

            lax.fori_loop(0, 16, score, 0)
            pltpu.async_copy(krows.at[p], ex_h.at[pl.ds(eoff(i), 128)],
                             wsem.at[p])
            return c

        lax.fori_loop(0, nsup, body, 0)

        def drain(t, c):
            i = nsup - 2 + t
            p = lax.rem(i, 2)
            pltpu.make_async_copy(krows.at[p],
                                  ex_h.at[pl.ds(eoff(i), 128)],
                                  wsem.at[p]).wait()
            return c

        lax.fori_loop(0, 2, drain, 0)

    return k


def _sc_score(kq, q, src, dst, re):
    if re is None:
        return _sc_score_fn(False)(kq, q, src, dst)
    return _sc_score_fn(True)(kq, q, src, dst, re)


def _zero_rows(buf, nrows, Dv):
    zv = jnp.zeros((16,), jnp.float32)
    for r in range(nrows):
        for j in range(Dv // 16):
            buf[r, pl.ds(j * 16, 16)] = zv


@functools.cache
def _sc_scatter_add_fn(B, Dv):
    CH = 80
    rpw = B // NW
    nch = rpw // CH

    @functools.partial(
        pl.kernel,
        out_type=jax.ShapeDtypeStruct((2 * NP, Dv), jnp.float32),
        mesh=_mesh,
        scratch_types=[
            pltpu.VMEM((CH,), jnp.int32),
            pltpu.VMEM((CH, Dv), jnp.float32),
            pltpu.VMEM_SHARED((NP, Dv), jnp.float32),
            pltpu.SemaphoreType.DMA,
        ],
    )
    def k(vals_h, idx_h, out_h, idx_v, vals_v, acc, sem):
        cid = lax.axis_index("c")
        sid = lax.axis_index("s")
        _zero_rows(vals_v, CH, Dv)
        for i, (o, ln) in enumerate([(0, 80), (80, 80), (160, 80), (240, 80),
                                     (320, 80), (400, 80), (480, 80),
                                     (560, 72)]):
            pltpu.sync_copy(vals_v.at[pl.ds(0, ln)],
                            acc.at[pl.ds(sid * RPS + o, ln)])
        plsc.subcore_barrier()

        base = _wid() * rpw

        def body(i, c):
            off = pl.multiple_of(base + i * CH, 8)
            pltpu.sync_copy(idx_h.at[pl.ds(off, CH)], idx_v)
            pltpu.sync_copy(vals_h.at[pl.ds(off, CH)], vals_v)
            pltpu.sync_copy(vals_v, acc.at[idx_v], add=True)
            return c

        lax.fori_loop(0, nch, body, 0)
        plsc.subcore_barrier()
        pltpu.sync_copy(acc.at[pl.ds(sid * RPS, RPS)],
                        out_h.at[pl.ds(cid * NP + sid * RPS, RPS)])

    return k


def _sc_scatter_add(vals, idx):
    return _sc_scatter_add_fn(vals.shape[0], vals.shape[1])(vals, idx)


@functools.cache
def _sc_hop_fused_fn():
    @functools.partial(
        pl.kernel,
        out_type=jax.ShapeDtypeStruct((2 * NP, D), jnp.float32),
        mesh=_mesh,
        scratch_types=[
            pltpu.VMEM((2, 128), jnp.int32),
            pltpu.VMEM((8, 128), jnp.int32),
            pltpu.VMEM((128, D), jnp.float32),
            pltpu.VMEM((128, D), jnp.float32),
            pltpu.VMEM_SHARED((NP, D), jnp.float32),
            pltpu.SemaphoreType.DMA((2,)),
            pltpu.SemaphoreType.DMA,
            pltpu.SemaphoreType.DMA,
            pltpu.SemaphoreType.DMA,
            pltpu.SemaphoreType.DMA,
        ],
    )
    def k(feat_h, att_h, src_h, dst2_h, out_h,
          sidx, didx, attv, fev, acc,
          ssem, dsem, asem, gsem, wsem):
        cid = lax.axis_index("c")
        sid = lax.axis_index("s")
        w = _wid()
        base = w * RW
        rbase = w * (RW // 128)
        nsup = _nsup(w, 128)

        _zero_rows(fev, 128, D)
        for o, ln in [(0, 128), (128, 128), (256, 128), (384, 128),
                      (512, 120)]:
            pltpu.sync_copy(fev.at[pl.ds(0, ln)],
                            acc.at[pl.ds(sid * RPS + o, ln)])
        plsc.subcore_barrier()

        def eoff(i):
            return pl.multiple_of(base + i * 128, 128)

        def load_sidx(i):
            pltpu.async_copy(src_h.at[pl.ds(eoff(i), 128)],
                             sidx.at[lax.rem(i, 2)], ssem.at[lax.rem(i, 2)])

        def load_dblock(i):
            pltpu.async_copy(
                dst2_h.at[pl.ds(pl.multiple_of(rbase + i, 8), 8)], didx, dsem)

        def load_att(i):
            pltpu.async_copy(att_h.at[pl.ds(eoff(i), 128)], attv, asem)

        def wait_sidx(i):
            pltpu.make_async_copy(src_h.at[pl.ds(base, 128)],
                                  sidx.at[lax.rem(i, 2)],
                                  ssem.at[lax.rem(i, 2)]).wait()

        def wait_scatter():
            pltpu.make_async_copy(fev, acc.at[didx.at[0]], wsem).wait()

        load_sidx(0)
        load_sidx(1)
        load_dblock(0)
        load_att(0)

        def body(i, c):
            p = lax.rem(i, 2)

            @pl.when(i >= 1)
            def _():
                wait_scatter()
            wait_sidx(i)
            pltpu.async_copy(feat_h.at[sidx.at[p]], fev, gsem)

            @pl.when(jnp.logical_and(lax.rem(i, 8) == 0, i > 0))
            def _():
                load_dblock(i)

            pltpu.make_async_copy(feat_h.at[sidx.at[p]], fev, gsem).wait()

            @pl.when(i + 2 < nsup)
            def _():
                load_sidx(i + 2)
            pltpu.make_async_copy(att_h.at[pl.ds(base, 128)], attv,
                                  asem).wait()

            def mul(g, c2):
                rb = pl.multiple_of(g * 8, 8)
                for rs in range(8):
                    r = rb + rs
                    for j in range(8):
                        sl = pl.ds(16 * j, 16)
                        fev[r, sl] = fev[r, sl] * attv[r, sl]
                return c2

            lax.fori_loop(0, 16, mul, 0)

            @pl.when(i + 1 < nsup)
            def _():
                load_att(i + 1)

            @pl.when(lax.rem(i, 8) == 0)
            def _():
                pltpu.make_async_copy(dst2_h.at[pl.ds(rbase, 8)], didx,
                                      dsem).wait()

            pltpu.async_copy(fev, acc.at[didx.at[lax.rem(i, 8)]], wsem,
                             add=True)
            return c

        lax.fori_loop(0, nsup, body, 0)
        wait_scatter()
        plsc.subcore_barrier()
        pltpu.sync_copy(acc.at[pl.ds(sid * RPS, RPS)],
                        out_h.at[pl.ds(cid * NP + sid * RPS, RPS)])

    return k


def _sc_hop_fused(feat, att, src, dst2):
    return _sc_hop_fused_fn()(feat, att, src, dst2)


def _tc_qkv(h, Wt, Wh, Wv):
    BN = 2528

    def body(h_r, wt_r, wh_r, wv_r, q_r, k_r, v_r):
        hb = h_r[...]
        q_r[...] = jnp.dot(hb, wt_r[...],
                           preferred_element_type=jnp.float32) * 0.25
        k_r[...] = jnp.dot(hb, wh_r[...], preferred_element_type=jnp.float32)
        v_r[...] = jnp.dot(hb, wv_r[...], preferred_element_type=jnp.float32)

    bs_h = pl.BlockSpec((BN, D), lambda i: (i, 0))
    bs_w = pl.BlockSpec((D, D), lambda i: (0, 0))
    return pl.pallas_call(
        body, grid=(NP // BN,),
        in_specs=[bs_h, bs_w, bs_w, bs_w],
        out_specs=[bs_h, bs_h, bs_h],
        out_shape=[jax.ShapeDtypeStruct((NP, D), jnp.float32)] * 3,
    )(h, Wt, Wh, Wv)


def _tc_relproj(rel_emb, Wr):
    def body(re_r, wr_r, o_r):
        o_r[...] = jnp.dot(re_r[...], wr_r[...],
                           preferred_element_type=jnp.float32)

    return pl.pallas_call(
        body,
        out_shape=jax.ShapeDtypeStruct((16, D), jnp.float32),
    )(rel_emb, Wr)


def _tc_tile_r(R):
    def body(r_r, o_r):
        o_r[...] = r_r[...]

    bs = pl.BlockSpec((16, D), lambda i: (0, 0))
    bo = pl.BlockSpec((16, D), lambda i: (i, 0))
    return pl.pallas_call(
        body, grid=(128,), in_specs=[bs], out_specs=bo,
        out_shape=jax.ShapeDtypeStruct((2048, D), jnp.float32),
    )(R)


def _tc_rel2(rel2d):
    BE = 2500

    def body(r_r, o_r):
        o_r[...] = r_r[...] + 16 * jax.lax.broadcasted_iota(
            jnp.int32, (BE, 128), 1)

    bs = pl.BlockSpec((BE, 128), lambda i: (i, 0))
    return pl.pallas_call(
        body, grid=(2500 // BE,), in_specs=[bs], out_specs=bs,
        out_shape=jax.ShapeDtypeStruct((2500, 128), jnp.int32),
    )(rel2d)


def _tc_exps(kq, S):
    BE = 4000
    bs_e = pl.BlockSpec((BE, D), lambda i: (i, 0))
    bs_s = pl.BlockSpec((D, D), lambda i: (0, 0))

    def body(kq_r, s_r, ex_r):
        ex_r[...] = jnp.exp(
            jnp.dot(kq_r[...], s_r[...], preferred_element_type=jnp.float32))

    return pl.pallas_call(
        body, grid=(E // BE,), in_specs=[bs_e, bs_s], out_specs=bs_e,
        out_shape=jax.ShapeDtypeStruct((E, D), jnp.float32),
    )(kq, S)


def _tc_sddmm(ke, qe, re, S):
    BE = 4000
    bs_e = pl.BlockSpec((BE, D), lambda i: (i, 0))
    bs_s = pl.BlockSpec((D, D), lambda i: (0, 0))
    if re is None:
        def body(ke_r, qe_r, s_r, ex_r):
            t = ke_r[...] * qe_r[...]
            ex_r[...] = jnp.exp(
                jnp.dot(t, s_r[...], preferred_element_type=jnp.float32))

        ins, specs = (ke, qe, S), [bs_e, bs_e, bs_s]
    else:
        def body(ke_r, re_r, qe_r, s_r, ex_r):
            t = (ke_r[...] + re_r[...]) * qe_r[...]
            ex_r[...] = jnp.exp(
                jnp.dot(t, s_r[...], preferred_element_type=jnp.float32))

        ins, specs = (ke, re, qe, S), [bs_e, bs_e, bs_e, bs_s]
    return pl.pallas_call(
        body, grid=(E // BE,), in_specs=specs, out_specs=bs_e,
        out_shape=jax.ShapeDtypeStruct((E, D), jnp.float32),
    )(*ins)


def _tc_add(a, b):
    BN = 2528

    def body(a_r, b_r, o_r):
        o_r[...] = a_r[...] + b_r[...]

    bs = pl.BlockSpec((BN, a.shape[1]), lambda i: (i, 0))
    return pl.pallas_call(
        body, grid=(a.shape[0] // BN,), in_specs=[bs, bs], out_specs=bs,
        out_shape=jax.ShapeDtypeStruct(a.shape, jnp.float32),
    )(a, b)


def _tc_att(ex, den_e):
    BE = 4000

    def body(ex_r, de_r, o_r):
        o_r[...] = ex_r[...] / (de_r[...] + 1e-9)

    bs = pl.BlockSpec((BE, D), lambda i: (i, 0))
    return pl.pallas_call(
        body, grid=(E // BE,), in_specs=[bs, bs], out_specs=bs,
        out_shape=jax.ShapeDtypeStruct((E, D), jnp.float32),
    )(ex, den_e)


def _tc_combine(p0, p1, feat0):
    BN = 2528

    def body(p0_r, p1_r, f0_r, o_r):
        o_r[...] = ALPHA * f0_r[...] + (1.0 - ALPHA) * (p0_r[...] + p1_r[...])

    bs = pl.BlockSpec((BN, D), lambda i: (i, 0))
    return pl.pallas_call(
        body, grid=(NP // BN,), in_specs=[bs, bs, bs], out_specs=bs,
        out_shape=jax.ShapeDtypeStruct((NP, D), jnp.float32),
    )(p0, p1, feat0)


def _tc_layerout(p0, p1, feat0, h):
    BN = 2528

    def body(p0_r, p1_r, f0_r, h_r, o_r):
        f = ALPHA * f0_r[...] + (1.0 - ALPHA) * (p0_r[...] + p1_r[...])
        o_r[...] = jnp.where(f > 0, f, jnp.exp(f) - 1.0) + h_r[...]

    bs = pl.BlockSpec((BN, D), lambda i: (i, 0))
    return pl.pallas_call(
        body, grid=(NP // BN,), in_specs=[bs, bs, bs, bs], out_specs=bs,
        out_shape=jax.ShapeDtypeStruct((NP, D), jnp.float32),
    )(p0, p1, feat0, h)


def kernel(ent_ids, rel_ids, edge_index, batch_cls, node_emb, rel_emb, params):
    S = jnp.asarray(_S128_np)
    src = edge_index[0]
    dst = edge_index[1]
    dst2 = jnp.reshape(dst, (E // 128, 128))
    ent_p = jnp.concatenate(
        [ent_ids, jnp.zeros((NG - N,), ent_ids.dtype)])

    h = _sc_gather(node_emb, ent_p, 80)[:NP]
    R = _tc_relproj(rel_emb, params['l0_Wr'])
    rel2 = jnp.reshape(
        _tc_rel2(jnp.reshape(rel_ids, (E // 128, 128))), (E,))
    re_full = _sc_gather_big(_tc_tile_r(R), rel2)

    for l in range(3):
        p = params
        q, k, v = _tc_qkv(h, p[f'l{l}_Wt'], p[f'l{l}_Wh'], p[f'l{l}_Wv'])
        kq = _sc_score(k, q, src, dst, re_full if l == 0 else None)
        ex = _tc_exps(kq, S)
        dpart = _sc_scatter_add(ex, dst)
        denom = _tc_add(dpart[:NP], dpart[NP:])
        den_e = _sc_gather_big(denom, dst)
        att = _tc_att(ex, den_e)
        feat = v
        for hop in range(HOP):
            apart = _sc_hop_fused(feat, att, src, dst2)
            if hop < HOP - 1:
                feat = _tc_combine(apart[:NP], apart[NP:], v)
            else:
                h = _tc_layerout(apart[:NP], apart[NP:], v, h)

    return _sc_gather(h, batch_cls, 8)


# speedup vs baseline: 1.2137x; 1.1412x over previous
import functools

import numpy as np
import jax
import jax.numpy as jnp
from jax import lax
from jax.experimental import pallas as pl
from jax.experimental.pallas import tpu as pltpu
from jax.experimental.pallas import tpu_sc as plsc

N = 10000
E = 320000
D = 128
H = 8
HD = 16
HOP = 4
ALPHA = 0.15
NP = 10112
NG = 10240
NC = 2
NS = 16
NW = NC * NS
RPS = NP // NS

RW = 10240

_mesh = plsc.VectorSubcoreMesh(core_axis_name="c", subcore_axis_name="s")

_S128_np = np.zeros((D, D), np.float32)
for _a in range(D):
    for _b in range(D):
        if _a // HD == _b // HD:
            _S128_np[_a, _b] = 1.0


def _wid():
    return lax.axis_index("s") * NC + lax.axis_index("c")


def _nsup(w, sup):
    return jnp.where(w == NW - 1, (RW // 4) // sup, RW // sup)


@functools.cache
def _sc_gather_fn(T, Dv, B, chunk):
    rpw = B // NW
    nch = rpw // chunk

    @functools.partial(
        pl.kernel,
        out_type=jax.ShapeDtypeStruct((B, Dv), jnp.float32),
        mesh=_mesh,
        scratch_types=[
            pltpu.VMEM((chunk,), jnp.int32),
            pltpu.VMEM((chunk, Dv), jnp.float32),
            pltpu.SemaphoreType.DMA,
        ],
    )
    def k(table_h, idx_h, out_h, idx_v, rows_v, sem):
        base = _wid() * rpw

        def body(i, c):
            off = pl.multiple_of(base + i * chunk, 8)
            pltpu.sync_copy(idx_h.at[pl.ds(off, chunk)], idx_v)
            pltpu.async_copy(table_h.at[idx_v], rows_v, sem).wait()
            pltpu.sync_copy(rows_v, out_h.at[pl.ds(off, chunk)])
            return c

        lax.fori_loop(0, nch, body, 0)

    return k


def _sc_gather(table, idx, chunk):
    return _sc_gather_fn(table.shape[0], table.shape[1], idx.shape[0], chunk)(
        table, idx)


_SUP = 256


@functools.cache
def _sc_gather_big_fn(T, Dv):
    @functools.partial(
        pl.kernel,
        out_type=jax.ShapeDtypeStruct((E, Dv), jnp.float32),
        mesh=_mesh,
        scratch_types=[
            pltpu.VMEM((2, _SUP), jnp.int32),
            pltpu.VMEM((2, _SUP, Dv), jnp.float32),
            pltpu.SemaphoreType.DMA((2,)),
            pltpu.SemaphoreType.DMA((2,)),
            pltpu.SemaphoreType.DMA((2,)),
        ],
    )
    def k(table_h, idx_h, out_h, idx_v, rows_v, isem, gsem, wsem):
        w = _wid()
        base = w * RW
        nsup = _nsup(w, _SUP)

        def off(i):
            return pl.multiple_of(base + i * _SUP, _SUP)

        def idx_load(i, p):
            pltpu.async_copy(idx_h.at[pl.ds(off(i), _SUP)], idx_v.at[p],
                             isem.at[p])

        idx_load(0, 0)

        def body(i, c):
            p = lax.rem(i, 2)

            @pl.when(i + 1 < nsup)
            def _():
                idx_load(i + 1, 1 - p)

            pltpu.make_async_copy(idx_h.at[pl.ds(off(i), _SUP)], idx_v.at[p],
                                  isem.at[p]).wait()

            @pl.when(i >= 2)
            def _():
                pltpu.make_async_copy(
                    rows_v.at[p], out_h.at[pl.ds(off(i - 2), _SUP)],
                    wsem.at[p]).wait()

            for j in range(_SUP // 128):
                pltpu.async_copy(
                    table_h.at[idx_v.at[p, pl.ds(j * 128, 128)]],
                    rows_v.at[p, pl.ds(j * 128, 128)], gsem.at[p])
            for j in range(_SUP // 128):
                pltpu.make_async_copy(
                    table_h.at[idx_v.at[p, pl.ds(j * 128, 128)]],
                    rows_v.at[p, pl.ds(j * 128, 128)], gsem.at[p]).wait()
            pltpu.async_copy(rows_v.at[p], out_h.at[pl.ds(off(i), _SUP)],
                             wsem.at[p])
            return c

        lax.fori_loop(0, nsup, body, 0)

        def drain(t, c):
            i = nsup - 2 + t

            @pl.when(i >= 0)
            def _():
                p = lax.rem(i, 2)
                pltpu.make_async_copy(
                    rows_v.at[p], out_h.at[pl.ds(off(i), _SUP)],
                    wsem.at[p]).wait()
            return c

        lax.fori_loop(0, 2, drain, 0)

    return k


def _sc_gather_big(table, idx):
    return _sc_gather_big_fn(table.shape[0], table.shape[1])(table, idx)


@functools.cache
def _sc_score_fn(with_rel):
    scratch = [
        pltpu.VMEM((2, 128), jnp.int32),
        pltpu.VMEM((2, 128), jnp.int32),
        pltpu.VMEM((2, 128, D), jnp.float32),
        pltpu.VMEM((2, 128, D), jnp.float32),
        pltpu.SemaphoreType.DMA((2,)),
        pltpu.SemaphoreType.DMA((2,)),
        pltpu.SemaphoreType.DMA((2,)),
    ]
    if with_rel:
        scratch.insert(4, pltpu.VMEM((2, 128, D), jnp.float32))

    @functools.partial(
        pl.kernel,
        out_type=jax.ShapeDtypeStruct((E, D), jnp.float32),
        mesh=_mesh,
        scratch_types=scratch,
    )
    def k(*args):
        if with_rel:
            (k_h, q_h, src_h, dst_h, re_h, ex_h,
             sidx, didx, krows, qrows, rerows, isem, gsem, wsem) = args
        else:
            (k_h, q_h, src_h, dst_h, ex_h,
             sidx, didx, krows, qrows, isem, gsem, wsem) = args
            rerows = None
        w = _wid()
        base = w * RW
        nsup = _nsup(w, 128)

        def eoff(i):
            return pl.multiple_of(base + i * 128, 128)

        def idx_load(i, p):
            pltpu.async_copy(src_h.at[pl.ds(eoff(i), 128)], sidx.at[p],
                             isem.at[p])
            pltpu.async_copy(dst_h.at[pl.ds(eoff(i), 128)], didx.at[p],
                             isem.at[p])

        def wait_idx(i, p):
            pltpu.make_async_copy(src_h.at[pl.ds(base, 128)], sidx.at[p],
                                  isem.at[p]).wait()
            pltpu.make_async_copy(dst_h.at[pl.ds(base, 128)], didx.at[p],
                                  isem.at[p]).wait()

        def gathers(i, p):
            pltpu.async_copy(k_h.at[sidx.at[p]], krows.at[p], gsem.at[p])
            pltpu.async_copy(q_h.at[didx.at[p]], qrows.at[p], gsem.at[p])
            if with_rel:
                pltpu.async_copy(re_h.at[pl.ds(eoff(i), 128)], rerows.at[p],
                                 gsem.at[p])

        def wait_gathers(i, p):
            pltpu.make_async_copy(k_h.at[sidx.at[p]], krows.at[p],
                                  gsem.at[p]).wait()
            pltpu.make_async_copy(q_h.at[didx.at[p]], qrows.at[p],
                                  gsem.at[p]).wait()
            if with_rel:
                pltpu.make_async_copy(re_h.at[pl.ds(base, 128)], rerows.at[p],
                                      gsem.at[p]).wait()

        idx_load(0, 0)
        wait_idx(0, 0)
        gathers(0, 0)
        idx_load(1, 1)

        def body(i, c):
            p = lax.rem(i, 2)

            @pl.when(i + 1 < nsup)
            def _():
                wait_idx(i + 1, 1 - p)

                @pl.when(i >= 1)
                def _():
                    pltpu.make_async_copy(
                        krows.at[1 - p], ex_h.at[pl.ds(eoff(i - 1), 128)],
                        wsem.at[1 - p]).wait()
                gathers(i + 1, 1 - p)

            wait_gathers(i, p)

            @pl.when(i + 2 < nsup)
            def _():
                idx_load(i + 2, p)

            def score(g, c2):
                rb = pl.multiple_of(g * 8, 8)
                for rs in range(8):
                    r = rb + rs
                    for j in range(8):
                        sl = pl.ds(16 * j, 16)
                        kv = krows[p, r, sl]
                        if with_rel:
                            kv = kv + rerows[p, r, sl]
                        krows[p, r, sl] = kv * qrows[p, r, sl]
                return c2

            lax.fori_loop(0, 16, score, 0)
            pltpu.async_copy(krows.at[p], ex_h.at[pl.ds(eoff(i), 128)],
                             wsem.at[p])
            return c

        lax.fori_loop(0, nsup, body, 0)

        def drain(t, c):
            i = nsup - 2 + t
            p = lax.rem(i, 2)
            pltpu.make_async_copy(krows.at[p],
                                  ex_h.at[pl.ds(eoff(i), 128)],
                                  wsem.at[p]).wait()
            return c

        lax.fori_loop(0, 2, drain, 0)

    return k


def _sc_score(kq, q, src, dst, re):
    if re is None:
        return _sc_score_fn(False)(kq, q, src, dst)
    return _sc_score_fn(True)(kq, q, src, dst, re)


def _zero_rows(buf, nrows, Dv):
    zv = jnp.zeros((16,), jnp.float32)
    for r in range(nrows):
        for j in range(Dv // 16):
            buf[r, pl.ds(j * 16, 16)] = zv


@functools.cache
def _sc_scatter_add_fn():
    @functools.partial(
        pl.kernel,
        out_type=jax.ShapeDtypeStruct((2 * NP, D), jnp.float32),
        mesh=_mesh,
        scratch_types=[
            pltpu.VMEM((8, 128), jnp.int32),
            pltpu.VMEM((2, 128, D), jnp.float32),
            pltpu.VMEM_SHARED((NP, D), jnp.float32),
            pltpu.SemaphoreType.DMA,
            pltpu.SemaphoreType.DMA((2,)),
            pltpu.SemaphoreType.DMA((2,)),
        ],
    )
    def k(vals_h, dst2_h, out_h, didx, vv, acc, dsem, vsem, wsem):
        cid = lax.axis_index("c")
        sid = lax.axis_index("s")
        w = _wid()
        base = w * RW
        rbase = w * (RW // 128)
        nsup = _nsup(w, 128)

        _zero_rows(vv.at[0], 128, D)
        for o, ln in [(0, 128), (128, 128), (256, 128), (384, 128),
                      (512, 120)]:
            pltpu.sync_copy(vv.at[0, pl.ds(0, ln)],
                            acc.at[pl.ds(sid * RPS + o, ln)])
        plsc.subcore_barrier()

        def eoff(i):
            return pl.multiple_of(base + i * 128, 128)

        def load_dblock(i):
            pltpu.async_copy(
                dst2_h.at[pl.ds(pl.multiple_of(rbase + i, 8), 8)], didx, dsem)

        def load_vals(i):
            pltpu.async_copy(vals_h.at[pl.ds(eoff(i), 128)],
                             vv.at[lax.rem(i, 2)], vsem.at[lax.rem(i, 2)])

        def wait_vals(i):
            pltpu.make_async_copy(vals_h.at[pl.ds(base, 128)],
                                  vv.at[lax.rem(i, 2)],
                                  vsem.at[lax.rem(i, 2)]).wait()

        def wait_scatter(i):
            p = lax.rem(i, 2)
            pltpu.make_async_copy(vv.at[p], acc.at[didx.at[0]],
                                  wsem.at[p]).wait()

        load_dblock(0)
        load_vals(0)

        def body(i, c):
            p = lax.rem(i, 2)

            @pl.when(i >= 1)
            def _():
                wait_scatter(i - 1)

            @pl.when(jnp.logical_and(lax.rem(i, 8) == 0, i > 0))
            def _():
                load_dblock(i)

            @pl.when(lax.rem(i, 8) == 0)
            def _():
                pltpu.make_async_copy(dst2_h.at[pl.ds(rbase, 8)], didx,
                                      dsem).wait()
            wait_vals(i)

            @pl.when(i + 1 < nsup)
            def _():
                load_vals(i + 1)

            pltpu.async_copy(vv.at[p], acc.at[didx.at[lax.rem(i, 8)]],
                             wsem.at[p], add=True)
            return c

        lax.fori_loop(0, nsup, body, 0)
        wait_scatter(nsup - 1)
        plsc.subcore_barrier()
        pltpu.sync_copy(acc.at[pl.ds(sid * RPS, RPS)],
                        out_h.at[pl.ds(cid * NP + sid * RPS, RPS)])

    return k


def _sc_scatter_add(vals, dst2):
    return _sc_scatter_add_fn()(vals, dst2)


@functools.cache
def _sc_hop_fused_fn():
    @functools.partial(
        pl.kernel,
        out_type=jax.ShapeDtypeStruct((2 * NP, D), jnp.float32),
        mesh=_mesh,
        scratch_types=[
            pltpu.VMEM((2, 128), jnp.int32),
            pltpu.VMEM((8, 128), jnp.int32),
            pltpu.VMEM((128, D), jnp.float32),
            pltpu.VMEM((128, D), jnp.float32),
            pltpu.VMEM_SHARED((NP, D), jnp.float32),
            pltpu.SemaphoreType.DMA((2,)),
            pltpu.SemaphoreType.DMA,
            pltpu.SemaphoreType.DMA,
            pltpu.SemaphoreType.DMA,
            pltpu.SemaphoreType.DMA,
        ],
    )
    def k(feat_h, att_h, src_h, dst2_h, out_h,
          sidx, didx, attv, fev, acc,
          ssem, dsem, asem, gsem, wsem):
        cid = lax.axis_index("c")
        sid = lax.axis_index("s")
        w = _wid()
        base = w * RW
        rbase = w * (RW // 128)
        nsup = _nsup(w, 128)

        _zero_rows(fev, 128, D)
        for o, ln in [(0, 128), (128, 128), (256, 128), (384, 128),
                      (512, 120)]:
            pltpu.sync_copy(fev.at[pl.ds(0, ln)],
                            acc.at[pl.ds(sid * RPS + o, ln)])
        plsc.subcore_barrier()

        def eoff(i):
            return pl.multiple_of(base + i * 128, 128)

        def load_sidx(i):
            pltpu.async_copy(src_h.at[pl.ds(eoff(i), 128)],
                             sidx.at[lax.rem(i, 2)], ssem.at[lax.rem(i, 2)])

        def load_dblock(i):
            pltpu.async_copy(
                dst2_h.at[pl.ds(pl.multiple_of(rbase + i, 8), 8)], didx, dsem)

        def load_att(i):
            pltpu.async_copy(att_h.at[pl.ds(eoff(i), 128)], attv, asem)

        def wait_sidx(i):
            pltpu.make_async_copy(src_h.at[pl.ds(base, 128)],
                                  sidx.at[lax.rem(i, 2)],
                                  ssem.at[lax.rem(i, 2)]).wait()

        def wait_scatter():
            pltpu.make_async_copy(fev, acc.at[didx.at[0]], wsem).wait()

        load_sidx(0)
        load_sidx(1)
        load_dblock(0)
        load_att(0)

        def body(i, c):
            p = lax.rem(i, 2)

            @pl.when(i >= 1)
            def _():
                wait_scatter()
            wait_sidx(i)
            pltpu.async_copy(feat_h.at[sidx.at[p]], fev, gsem)

            @pl.when(jnp.logical_and(lax.rem(i, 8) == 0, i > 0))
            def _():
                load_dblock(i)

            pltpu.make_async_copy(feat_h.at[sidx.at[p]], fev, gsem).wait()

            @pl.when(i + 2 < nsup)
            def _():
                load_sidx(i + 2)
            pltpu.make_async_copy(att_h.at[pl.ds(base, 128)], attv,
                                  asem).wait()

            def mul(g, c2):
                rb = pl.multiple_of(g * 8, 8)
                for rs in range(8):
                    r = rb + rs
                    for j in range(8):
                        sl = pl.ds(16 * j, 16)
                        fev[r, sl] = fev[r, sl] * attv[r, sl]
                return c2

            lax.fori_loop(0, 16, mul, 0)

            @pl.when(i + 1 < nsup)
            def _():
                load_att(i + 1)

            @pl.when(lax.rem(i, 8) == 0)
            def _():
                pltpu.make_async_copy(dst2_h.at[pl.ds(rbase, 8)], didx,
                                      dsem).wait()

            pltpu.async_copy(fev, acc.at[didx.at[lax.rem(i, 8)]], wsem,
                             add=True)
            return c

        lax.fori_loop(0, nsup, body, 0)
        wait_scatter()
        plsc.subcore_barrier()
        pltpu.sync_copy(acc.at[pl.ds(sid * RPS, RPS)],
                        out_h.at[pl.ds(cid * NP + sid * RPS, RPS)])

    return k


def _sc_hop_fused(feat, att, src, dst2):
    return _sc_hop_fused_fn()(feat, att, src, dst2)


def _tc_qkv(h, Wt, Wh, Wv):
    BN = 2528

    def body(h_r, wt_r, wh_r, wv_r, q_r, k_r, v_r):
        hb = h_r[...]
        q_r[...] = jnp.dot(hb, wt_r[...],
                           preferred_element_type=jnp.float32) * 0.25
        k_r[...] = jnp.dot(hb, wh_r[...], preferred_element_type=jnp.float32)
        v_r[...] = jnp.dot(hb, wv_r[...], preferred_element_type=jnp.float32)

    bs_h = pl.BlockSpec((BN, D), lambda i: (i, 0))
    bs_w = pl.BlockSpec((D, D), lambda i: (0, 0))
    return pl.pallas_call(
        body, grid=(NP // BN,),
        in_specs=[bs_h, bs_w, bs_w, bs_w],
        out_specs=[bs_h, bs_h, bs_h],
        out_shape=[jax.ShapeDtypeStruct((NP, D), jnp.float32)] * 3,
    )(h, Wt, Wh, Wv)


def _tc_relproj(rel_emb, Wr):
    def body(re_r, wr_r, o_r):
        o_r[...] = jnp.dot(re_r[...], wr_r[...],
                           preferred_element_type=jnp.float32)

    return pl.pallas_call(
        body,
        out_shape=jax.ShapeDtypeStruct((16, D), jnp.float32),
    )(rel_emb, Wr)


def _tc_tile_r(R):
    def body(r_r, o_r):
        o_r[...] = r_r[...]

    bs = pl.BlockSpec((16, D), lambda i: (0, 0))
    bo = pl.BlockSpec((16, D), lambda i: (i, 0))
    return pl.pallas_call(
        body, grid=(128,), in_specs=[bs], out_specs=bo,
        out_shape=jax.ShapeDtypeStruct((2048, D), jnp.float32),
    )(R)


def _tc_rel2(rel2d):
    BE = 2500

    def body(r_r, o_r):
        o_r[...] = r_r[...] + 16 * jax.lax.broadcasted_iota(
            jnp.int32, (BE, 128), 1)

    bs = pl.BlockSpec((BE, 128), lambda i: (i, 0))
    return pl.pallas_call(
        body, grid=(2500 // BE,), in_specs=[bs], out_specs=bs,
        out_shape=jax.ShapeDtypeStruct((2500, 128), jnp.int32),
    )(rel2d)


def _tc_exps(kq, S):
    BE = 4000
    bs_e = pl.BlockSpec((BE, D), lambda i: (i, 0))
    bs_s = pl.BlockSpec((D, D), lambda i: (0, 0))

    def body(kq_r, s_r, ex_r):
        ex_r[...] = jnp.exp(
            jnp.dot(kq_r[...], s_r[...], preferred_element_type=jnp.float32))

    return pl.pallas_call(
        body, grid=(E // BE,), in_specs=[bs_e, bs_s], out_specs=bs_e,
        out_shape=jax.ShapeDtypeStruct((E, D), jnp.float32),
    )(kq, S)


def _tc_sddmm(ke, qe, re, S):
    BE = 4000
    bs_e = pl.BlockSpec((BE, D), lambda i: (i, 0))
    bs_s = pl.BlockSpec((D, D), lambda i: (0, 0))
    if re is None:
        def body(ke_r, qe_r, s_r, ex_r):
            t = ke_r[...] * qe_r[...]
            ex_r[...] = jnp.exp(
                jnp.dot(t, s_r[...], preferred_element_type=jnp.float32))

        ins, specs = (ke, qe, S), [bs_e, bs_e, bs_s]
    else:
        def body(ke_r, re_r, qe_r, s_r, ex_r):
            t = (ke_r[...] + re_r[...]) * qe_r[...]
            ex_r[...] = jnp.exp(
                jnp.dot(t, s_r[...], preferred_element_type=jnp.float32))

        ins, specs = (ke, re, qe, S), [bs_e, bs_e, bs_e, bs_s]
    return pl.pallas_call(
        body, grid=(E // BE,), in_specs=specs, out_specs=bs_e,
        out_shape=jax.ShapeDtypeStruct((E, D), jnp.float32),
    )(*ins)


def _tc_add(a, b):
    BN = 2528

    def body(a_r, b_r, o_r):
        o_r[...] = a_r[...] + b_r[...]

    bs = pl.BlockSpec((BN, a.shape[1]), lambda i: (i, 0))
    return pl.pallas_call(
        body, grid=(a.shape[0] // BN,), in_specs=[bs, bs], out_specs=bs,
        out_shape=jax.ShapeDtypeStruct(a.shape, jnp.float32),
    )(a, b)


def _tc_att(ex, den_e):
    BE = 4000

    def body(ex_r, de_r, o_r):
        o_r[...] = ex_r[...] / (de_r[...] + 1e-9)

    bs = pl.BlockSpec((BE, D), lambda i: (i, 0))
    return pl.pallas_call(
        body, grid=(E // BE,), in_specs=[bs, bs], out_specs=bs,
        out_shape=jax.ShapeDtypeStruct((E, D), jnp.float32),
    )(ex, den_e)


def _tc_combine(p0, p1, feat0):
    BN = 2528

    def body(p0_r, p1_r, f0_r, o_r):
        o_r[...] = ALPHA * f0_r[...] + (1.0 - ALPHA) * (p0_r[...] + p1_r[...])

    bs = pl.BlockSpec((BN, D), lambda i: (i, 0))
    return pl.pallas_call(
        body, grid=(NP // BN,), in_specs=[bs, bs, bs], out_specs=bs,
        out_shape=jax.ShapeDtypeStruct((NP, D), jnp.float32),
    )(p0, p1, feat0)


def _tc_layerout(p0, p1, feat0, h):
    BN = 2528

    def body(p0_r, p1_r, f0_r, h_r, o_r):
        f = ALPHA * f0_r[...] + (1.0 - ALPHA) * (p0_r[...] + p1_r[...])
        o_r[...] = jnp.where(f > 0, f, jnp.exp(f) - 1.0) + h_r[...]

    bs = pl.BlockSpec((BN, D), lambda i: (i, 0))
    return pl.pallas_call(
        body, grid=(NP // BN,), in_specs=[bs, bs, bs, bs], out_specs=bs,
        out_shape=jax.ShapeDtypeStruct((NP, D), jnp.float32),
    )(p0, p1, feat0, h)


def kernel(ent_ids, rel_ids, edge_index, batch_cls, node_emb, rel_emb, params):
    S = jnp.asarray(_S128_np)
    src = edge_index[0]
    dst = edge_index[1]
    dst2 = jnp.reshape(dst, (E // 128, 128))
    ent_p = jnp.concatenate(
        [ent_ids, jnp.zeros((NG - N,), ent_ids.dtype)])

    h = _sc_gather(node_emb, ent_p, 80)[:NP]
    R = _tc_relproj(rel_emb, params['l0_Wr'])
    rel2 = jnp.reshape(
        _tc_rel2(jnp.reshape(rel_ids, (E // 128, 128))), (E,))
    re_full = _sc_gather_big(_tc_tile_r(R), rel2)

    for l in range(3):
        p = params
        q, k, v = _tc_qkv(h, p[f'l{l}_Wt'], p[f'l{l}_Wh'], p[f'l{l}_Wv'])
        ke = _sc_gather_big(k, src)
        qe = _sc_gather_big(q, dst)
        ex = _tc_sddmm(ke, qe, re_full if l == 0 else None, S)
        dpart = _sc_scatter_add(ex, dst2)
        denom = _tc_add(dpart[:NP], dpart[NP:])
        den_e = _sc_gather_big(denom, dst)
        att = _tc_att(ex, den_e)
        feat = v
        for hop in range(HOP):
            apart = _sc_hop_fused(feat, att, src, dst2)
            if hop < HOP - 1:
                feat = _tc_combine(apart[:NP], apart[NP:], v)
            else:
                h = _tc_layerout(apart[:NP], apart[NP:], v, h)

    return _sc_gather(h, batch_cls, 8)


# speedup vs baseline: 1.3816x; 1.1383x over previous
import functools

import numpy as np
import jax
import jax.numpy as jnp
from jax import lax
from jax.experimental import pallas as pl
from jax.experimental.pallas import tpu as pltpu
from jax.experimental.pallas import tpu_sc as plsc

N = 10000
E = 320000
D = 128
H = 8
HD = 16
HOP = 4
ALPHA = 0.15
NP = 10112
NG = 10240
NC = 2
NS = 16
NW = NC * NS
RPS = NP // NS

RW = 10240

_mesh = plsc.VectorSubcoreMesh(core_axis_name="c", subcore_axis_name="s")

_S128_np = np.zeros((D, D), np.float32)
for _a in range(D):
    for _b in range(D):
        if _a // HD == _b // HD:
            _S128_np[_a, _b] = 1.0


def _wid():
    return lax.axis_index("s") * NC + lax.axis_index("c")


def _nsup(w, sup):
    return jnp.where(w == NW - 1, (RW // 4) // sup, RW // sup)


@functools.cache
def _sc_gather_fn(T, Dv, B, chunk):
    rpw = B // NW
    nch = rpw // chunk

    @functools.partial(
        pl.kernel,
        out_type=jax.ShapeDtypeStruct((B, Dv), jnp.float32),
        mesh=_mesh,
        scratch_types=[
            pltpu.VMEM((chunk,), jnp.int32),
            pltpu.VMEM((chunk, Dv), jnp.float32),
            pltpu.SemaphoreType.DMA,
        ],
    )
    def k(table_h, idx_h, out_h, idx_v, rows_v, sem):
        base = _wid() * rpw

        def body(i, c):
            off = pl.multiple_of(base + i * chunk, 8)
            pltpu.sync_copy(idx_h.at[pl.ds(off, chunk)], idx_v)
            pltpu.async_copy(table_h.at[idx_v], rows_v, sem).wait()
            pltpu.sync_copy(rows_v, out_h.at[pl.ds(off, chunk)])
            return c

        lax.fori_loop(0, nch, body, 0)

    return k


def _sc_gather(table, idx, chunk):
    return _sc_gather_fn(table.shape[0], table.shape[1], idx.shape[0], chunk)(
        table, idx)


_SUP = 256


@functools.cache
def _sc_gather_big_fn(T, Dv):
    @functools.partial(
        pl.kernel,
        out_type=jax.ShapeDtypeStruct((E, Dv), jnp.float32),
        mesh=_mesh,
        scratch_types=[
            pltpu.VMEM((2, _SUP), jnp.int32),
            pltpu.VMEM((2, _SUP, Dv), jnp.float32),
            pltpu.SemaphoreType.DMA((2,)),
            pltpu.SemaphoreType.DMA((2,)),
            pltpu.SemaphoreType.DMA((2,)),
        ],
    )
    def k(table_h, idx_h, out_h, idx_v, rows_v, isem, gsem, wsem):
        w = _wid()
        base = w * RW
        nsup = _nsup(w, _SUP)

        def off(i):
            return pl.multiple_of(base + i * _SUP, _SUP)

        def idx_load(i, p):
            pltpu.async_copy(idx_h.at[pl.ds(off(i), _SUP)], idx_v.at[p],
                             isem.at[p])

        idx_load(0, 0)

        def body(i, c):
            p = lax.rem(i, 2)

            @pl.when(i + 1 < nsup)
            def _():
                idx_load(i + 1, 1 - p)

            pltpu.make_async_copy(idx_h.at[pl.ds(off(i), _SUP)], idx_v.at[p],
                                  isem.at[p]).wait()

            @pl.when(i >= 2)
            def _():
                pltpu.make_async_copy(
                    rows_v.at[p], out_h.at[pl.ds(off(i - 2), _SUP)],
                    wsem.at[p]).wait()

            for j in range(_SUP // 128):
                pltpu.async_copy(
                    table_h.at[idx_v.at[p, pl.ds(j * 128, 128)]],
                    rows_v.at[p, pl.ds(j * 128, 128)], gsem.at[p])
            for j in range(_SUP // 128):
                pltpu.make_async_copy(
                    table_h.at[idx_v.at[p, pl.ds(j * 128, 128)]],
                    rows_v.at[p, pl.ds(j * 128, 128)], gsem.at[p]).wait()
            pltpu.async_copy(rows_v.at[p], out_h.at[pl.ds(off(i), _SUP)],
                             wsem.at[p])
            return c

        lax.fori_loop(0, nsup, body, 0)

        def drain(t, c):
            i = nsup - 2 + t

            @pl.when(i >= 0)
            def _():
                p = lax.rem(i, 2)
                pltpu.make_async_copy(
                    rows_v.at[p], out_h.at[pl.ds(off(i), _SUP)],
                    wsem.at[p]).wait()
            return c

        lax.fori_loop(0, 2, drain, 0)

    return k


def _sc_gather_big(table, idx):
    return _sc_gather_big_fn(table.shape[0], table.shape[1])(table, idx)


@functools.cache
def _sc_score_fn(with_rel):
    scratch = [
        pltpu.VMEM((2, 128), jnp.int32),
        pltpu.VMEM((2, 128), jnp.int32),
        pltpu.VMEM((2, 128, D), jnp.float32),
        pltpu.VMEM((2, 128, D), jnp.float32),
        pltpu.SemaphoreType.DMA((2,)),
        pltpu.SemaphoreType.DMA((2,)),
        pltpu.SemaphoreType.DMA((2,)),
    ]
    if with_rel:
        scratch.insert(4, pltpu.VMEM((2, 128, D), jnp.float32))

    @functools.partial(
        pl.kernel,
        out_type=jax.ShapeDtypeStruct((E, D), jnp.float32),
        mesh=_mesh,
        scratch_types=scratch,
    )
    def k(*args):
        if with_rel:
            (k_h, q_h, src_h, dst_h, re_h, ex_h,
             sidx, didx, krows, qrows, rerows, isem, gsem, wsem) = args
        else:
            (k_h, q_h, src_h, dst_h, ex_h,
             sidx, didx, krows, qrows, isem, gsem, wsem) = args
            rerows = None
        w = _wid()
        base = w * RW
        nsup = _nsup(w, 128)

        def eoff(i):
            return pl.multiple_of(base + i * 128, 128)

        def idx_load(i, p):
            pltpu.async_copy(src_h.at[pl.ds(eoff(i), 128)], sidx.at[p],
                             isem.at[p])
            pltpu.async_copy(dst_h.at[pl.ds(eoff(i), 128)], didx.at[p],
                             isem.at[p])

        def wait_idx(i, p):
            pltpu.make_async_copy(src_h.at[pl.ds(base, 128)], sidx.at[p],
                                  isem.at[p]).wait()
            pltpu.make_async_copy(dst_h.at[pl.ds(base, 128)], didx.at[p],
                                  isem.at[p]).wait()

        def gathers(i, p):
            pltpu.async_copy(k_h.at[sidx.at[p]], krows.at[p], gsem.at[p])
            pltpu.async_copy(q_h.at[didx.at[p]], qrows.at[p], gsem.at[p])
            if with_rel:
                pltpu.async_copy(re_h.at[pl.ds(eoff(i), 128)], rerows.at[p],
                                 gsem.at[p])

        def wait_gathers(i, p):
            pltpu.make_async_copy(k_h.at[sidx.at[p]], krows.at[p],
                                  gsem.at[p]).wait()
            pltpu.make_async_copy(q_h.at[didx.at[p]], qrows.at[p],
                                  gsem.at[p]).wait()
            if with_rel:
                pltpu.make_async_copy(re_h.at[pl.ds(base, 128)], rerows.at[p],
                                      gsem.at[p]).wait()

        idx_load(0, 0)
        wait_idx(0, 0)
        gathers(0, 0)
        idx_load(1, 1)

        def body(i, c):
            p = lax.rem(i, 2)

            @pl.when(i + 1 < nsup)
            def _():
                wait_idx(i + 1, 1 - p)

                @pl.when(i >= 1)
                def _():
                    pltpu.make_async_copy(
                        krows.at[1 - p], ex_h.at[pl.ds(eoff(i - 1), 128)],
                        wsem.at[1 - p]).wait()
                gathers(i + 1, 1 - p)

            wait_gathers(i, p)

            @pl.when(i + 2 < nsup)
            def _():
                idx_load(i + 2, p)

            def score(g, c2):
                rb = pl.multiple_of(g * 8, 8)
                for rs in range(8):
                    r = rb + rs
                    for j in range(8):
                        sl = pl.ds(16 * j, 16)
                        kv = krows[p, r, sl]
                        if with_rel:
                            kv = kv + rerows[p, r, sl]
                        krows[p, r, sl] = kv * qrows[p, r, sl]
                return c2

            lax.fori_loop(0, 16, score, 0)
            pltpu.async_copy(krows.at[p], ex_h.at[pl.ds(eoff(i), 128)],
                             wsem.at[p])
            return c

        lax.fori_loop(0, nsup, body, 0)

        def drain(t, c):
            i = nsup - 2 + t
            p = lax.rem(i, 2)
            pltpu.make_async_copy(krows.at[p],
                                  ex_h.at[pl.ds(eoff(i), 128)],
                                  wsem.at[p]).wait()
            return c

        lax.fori_loop(0, 2, drain, 0)

    return k


def _sc_score(kq, q, src, dst, re):
    if re is None:
        return _sc_score_fn(False)(kq, q, src, dst)
    return _sc_score_fn(True)(kq, q, src, dst, re)


def _zero_rows(buf, nrows, Dv):
    zv = jnp.zeros((16,), jnp.float32)
    for r in range(nrows):
        for j in range(Dv // 16):
            buf[r, pl.ds(j * 16, 16)] = zv


@functools.cache
def _sc_scatter_add_fn():
    @functools.partial(
        pl.kernel,
        out_type=jax.ShapeDtypeStruct((2 * NP, D), jnp.float32),
        mesh=_mesh,
        scratch_types=[
            pltpu.VMEM((8, 128), jnp.int32),
            pltpu.VMEM((2, 128, D), jnp.float32),
            pltpu.VMEM_SHARED((NP, D), jnp.float32),
            pltpu.SemaphoreType.DMA,
            pltpu.SemaphoreType.DMA((2,)),
            pltpu.SemaphoreType.DMA((2,)),
        ],
    )
    def k(vals_h, dst2_h, out_h, didx, vv, acc, dsem, vsem, wsem):
        cid = lax.axis_index("c")
        sid = lax.axis_index("s")
        w = _wid()
        base = w * RW
        rbase = w * (RW // 128)
        nsup = _nsup(w, 128)

        _zero_rows(vv.at[0], 128, D)
        for o, ln in [(0, 128), (128, 128), (256, 128), (384, 128),
                      (512, 120)]:
            pltpu.sync_copy(vv.at[0, pl.ds(0, ln)],
                            acc.at[pl.ds(sid * RPS + o, ln)])
        plsc.subcore_barrier()

        def eoff(i):
            return pl.multiple_of(base + i * 128, 128)

        def load_dblock(i):
            pltpu.async_copy(
                dst2_h.at[pl.ds(pl.multiple_of(rbase + i, 8), 8)], didx, dsem)

        def load_vals(i):
            pltpu.async_copy(vals_h.at[pl.ds(eoff(i), 128)],
                             vv.at[lax.rem(i, 2)], vsem.at[lax.rem(i, 2)])

        def wait_vals(i):
            pltpu.make_async_copy(vals_h.at[pl.ds(base, 128)],
                                  vv.at[lax.rem(i, 2)],
                                  vsem.at[lax.rem(i, 2)]).wait()

        def wait_scatter(i):
            p = lax.rem(i, 2)
            pltpu.make_async_copy(vv.at[p], acc.at[didx.at[0]],
                                  wsem.at[p]).wait()

        load_dblock(0)
        load_vals(0)

        def body(i, c):
            p = lax.rem(i, 2)

            @pl.when(i >= 1)
            def _():
                wait_scatter(i - 1)

            @pl.when(jnp.logical_and(lax.rem(i, 8) == 0, i > 0))
            def _():
                load_dblock(i)

            @pl.when(lax.rem(i, 8) == 0)
            def _():
                pltpu.make_async_copy(dst2_h.at[pl.ds(rbase, 8)], didx,
                                      dsem).wait()
            wait_vals(i)

            @pl.when(i + 1 < nsup)
            def _():
                load_vals(i + 1)

            pltpu.async_copy(vv.at[p], acc.at[didx.at[lax.rem(i, 8)]],
                             wsem.at[p], add=True)
            return c

        lax.fori_loop(0, nsup, body, 0)
        wait_scatter(nsup - 1)
        plsc.subcore_barrier()
        pltpu.sync_copy(acc.at[pl.ds(sid * RPS, RPS)],
                        out_h.at[pl.ds(cid * NP + sid * RPS, RPS)])

    return k


def _sc_scatter_add(vals, dst2):
    return _sc_scatter_add_fn()(vals, dst2)


@functools.cache
def _sc_hop_fused_fn():
    @functools.partial(
        pl.kernel,
        out_type=jax.ShapeDtypeStruct((2 * NP, D), jnp.float32),
        mesh=_mesh,
        scratch_types=[
            pltpu.VMEM((2, 128), jnp.int32),
            pltpu.VMEM((8, 128), jnp.int32),
            pltpu.VMEM((128, D), jnp.float32),
            pltpu.VMEM((128, D), jnp.float32),
            pltpu.VMEM_SHARED((NP, D), jnp.float32),
            pltpu.SemaphoreType.DMA((2,)),
            pltpu.SemaphoreType.DMA,
            pltpu.SemaphoreType.DMA,
            pltpu.SemaphoreType.DMA,
            pltpu.SemaphoreType.DMA,
        ],
    )
    def k(feat_h, att_h, src_h, dst2_h, out_h,
          sidx, didx, attv, fev, acc,
          ssem, dsem, asem, gsem, wsem):
        cid = lax.axis_index("c")
        sid = lax.axis_index("s")
        w = _wid()
        base = w * RW
        rbase = w * (RW // 128)
        nsup = _nsup(w, 128)

        _zero_rows(fev, 128, D)
        for o, ln in [(0, 128), (128, 128), (256, 128), (384, 128),
                      (512, 120)]:
            pltpu.sync_copy(fev.at[pl.ds(0, ln)],
                            acc.at[pl.ds(sid * RPS + o, ln)])
        plsc.subcore_barrier()

        def eoff(i):
            return pl.multiple_of(base + i * 128, 128)

        def load_sidx(i):
            pltpu.async_copy(src_h.at[pl.ds(eoff(i), 128)],
                             sidx.at[lax.rem(i, 2)], ssem.at[lax.rem(i, 2)])

        def load_dblock(i):
            pltpu.async_copy(
                dst2_h.at[pl.ds(pl.multiple_of(rbase + i, 8), 8)], didx, dsem)

        def load_att(i):
            pltpu.async_copy(att_h.at[pl.ds(eoff(i), 128)], attv, asem)

        def wait_sidx(i):
            pltpu.make_async_copy(src_h.at[pl.ds(base, 128)],
                                  sidx.at[lax.rem(i, 2)],
                                  ssem.at[lax.rem(i, 2)]).wait()

        def wait_scatter():
            pltpu.make_async_copy(fev, acc.at[didx.at[0]], wsem).wait()

        load_sidx(0)
        load_sidx(1)
        load_dblock(0)
        load_att(0)

        def body(i, c):
            p = lax.rem(i, 2)

            @pl.when(i >= 1)
            def _():
                wait_scatter()
            wait_sidx(i)
            pltpu.async_copy(feat_h.at[sidx.at[p]], fev, gsem)

            @pl.when(jnp.logical_and(lax.rem(i, 8) == 0, i > 0))
            def _():
                load_dblock(i)

            pltpu.make_async_copy(feat_h.at[sidx.at[p]], fev, gsem).wait()

            @pl.when(i + 2 < nsup)
            def _():
                load_sidx(i + 2)
            pltpu.make_async_copy(att_h.at[pl.ds(base, 128)], attv,
                                  asem).wait()

            def mul(g, c2):
                rb = pl.multiple_of(g * 8, 8)
                for rs in range(8):
                    r = rb + rs
                    for j in range(8):
                        sl = pl.ds(16 * j, 16)
                        fev[r, sl] = fev[r, sl] * attv[r, sl]
                return c2

            lax.fori_loop(0, 16, mul, 0)

            @pl.when(i + 1 < nsup)
            def _():
                load_att(i + 1)

            @pl.when(lax.rem(i, 8) == 0)
            def _():
                pltpu.make_async_copy(dst2_h.at[pl.ds(rbase, 8)], didx,
                                      dsem).wait()

            pltpu.async_copy(fev, acc.at[didx.at[lax.rem(i, 8)]], wsem,
                             add=True)
            return c

        lax.fori_loop(0, nsup, body, 0)
        wait_scatter()
        plsc.subcore_barrier()
        pltpu.sync_copy(acc.at[pl.ds(sid * RPS, RPS)],
                        out_h.at[pl.ds(cid * NP + sid * RPS, RPS)])

    return k


def _sc_hop_fused(feat, att, src, dst2):
    return _sc_hop_fused_fn()(feat, att, src, dst2)


def _tc_qkv(h, Wt, Wh, Wv):
    BN = 2528

    def body(h_r, wt_r, wh_r, wv_r, q_r, k_r, v_r):
        hb = h_r[...]
        q_r[...] = jnp.dot(hb, wt_r[...],
                           preferred_element_type=jnp.float32) * 0.25
        k_r[...] = jnp.dot(hb, wh_r[...], preferred_element_type=jnp.float32)
        v_r[...] = jnp.dot(hb, wv_r[...], preferred_element_type=jnp.float32)

    bs_h = pl.BlockSpec((BN, D), lambda i: (i, 0))
    bs_w = pl.BlockSpec((D, D), lambda i: (0, 0))
    return pl.pallas_call(
        body, grid=(NP // BN,),
        in_specs=[bs_h, bs_w, bs_w, bs_w],
        out_specs=[bs_h, bs_h, bs_h],
        out_shape=[jax.ShapeDtypeStruct((NP, D), jnp.float32)] * 3,
    )(h, Wt, Wh, Wv)


def _tc_relproj(rel_emb, Wr):
    def body(re_r, wr_r, o_r):
        o_r[...] = jnp.dot(re_r[...], wr_r[...],
                           preferred_element_type=jnp.float32)

    return pl.pallas_call(
        body,
        out_shape=jax.ShapeDtypeStruct((16, D), jnp.float32),
    )(rel_emb, Wr)


def _tc_tile_r(R):
    def body(r_r, o_r):
        o_r[...] = r_r[...]

    bs = pl.BlockSpec((16, D), lambda i: (0, 0))
    bo = pl.BlockSpec((16, D), lambda i: (i, 0))
    return pl.pallas_call(
        body, grid=(128,), in_specs=[bs], out_specs=bo,
        out_shape=jax.ShapeDtypeStruct((2048, D), jnp.float32),
    )(R)


def _tc_rel2(rel2d):
    BE = 2500

    def body(r_r, o_r):
        o_r[...] = r_r[...] + 16 * jax.lax.broadcasted_iota(
            jnp.int32, (BE, 128), 1)

    bs = pl.BlockSpec((BE, 128), lambda i: (i, 0))
    return pl.pallas_call(
        body, grid=(2500 // BE,), in_specs=[bs], out_specs=bs,
        out_shape=jax.ShapeDtypeStruct((2500, 128), jnp.int32),
    )(rel2d)


def _tc_exps(kq, S):
    BE = 4000
    bs_e = pl.BlockSpec((BE, D), lambda i: (i, 0))
    bs_s = pl.BlockSpec((D, D), lambda i: (0, 0))

    def body(kq_r, s_r, ex_r):
        ex_r[...] = jnp.exp(
            jnp.dot(kq_r[...], s_r[...], preferred_element_type=jnp.float32))

    return pl.pallas_call(
        body, grid=(E // BE,), in_specs=[bs_e, bs_s], out_specs=bs_e,
        out_shape=jax.ShapeDtypeStruct((E, D), jnp.float32),
    )(kq, S)


def _tc_sddmm(ke, qe, re, S):
    BE = 4000
    bs_e = pl.BlockSpec((BE, D), lambda i: (i, 0))
    bs_s = pl.BlockSpec((D, D), lambda i: (0, 0))
    if re is None:
        def body(ke_r, qe_r, s_r, ex_r):
            t = ke_r[...] * qe_r[...]
            ex_r[...] = jnp.exp(
                jnp.dot(t, s_r[...], preferred_element_type=jnp.float32))

        ins, specs = (ke, qe, S), [bs_e, bs_e, bs_s]
    else:
        def body(ke_r, re_r, qe_r, s_r, ex_r):
            t = (ke_r[...] + re_r[...]) * qe_r[...]
            ex_r[...] = jnp.exp(
                jnp.dot(t, s_r[...], preferred_element_type=jnp.float32))

        ins, specs = (ke, re, qe, S), [bs_e, bs_e, bs_e, bs_s]
    return pl.pallas_call(
        body, grid=(E // BE,), in_specs=specs, out_specs=bs_e,
        out_shape=jax.ShapeDtypeStruct((E, D), jnp.float32),
    )(*ins)


def _tc_add(a, b):
    BN = 2528

    def body(a_r, b_r, o_r):
        o_r[...] = a_r[...] + b_r[...]

    bs = pl.BlockSpec((BN, a.shape[1]), lambda i: (i, 0))
    return pl.pallas_call(
        body, grid=(a.shape[0] // BN,), in_specs=[bs, bs], out_specs=bs,
        out_shape=jax.ShapeDtypeStruct(a.shape, jnp.float32),
    )(a, b)


def _tc_att(ex, den_e):
    BE = 4000

    def body(ex_r, de_r, o_r):
        o_r[...] = ex_r[...] / (de_r[...] + 1e-9)

    bs = pl.BlockSpec((BE, D), lambda i: (i, 0))
    return pl.pallas_call(
        body, grid=(E // BE,), in_specs=[bs, bs], out_specs=bs,
        out_shape=jax.ShapeDtypeStruct((E, D), jnp.float32),
    )(ex, den_e)


def _tc_combine(p0, p1, d0, d1, feat0):
    BN = 2528

    def body(p0_r, p1_r, d0_r, d1_r, f0_r, o_r):
        agg = (p0_r[...] + p1_r[...]) / (d0_r[...] + d1_r[...] + 1e-9)
        o_r[...] = ALPHA * f0_r[...] + (1.0 - ALPHA) * agg

    bs = pl.BlockSpec((BN, D), lambda i: (i, 0))
    return pl.pallas_call(
        body, grid=(NP // BN,), in_specs=[bs] * 5, out_specs=bs,
        out_shape=jax.ShapeDtypeStruct((NP, D), jnp.float32),
    )(p0, p1, d0, d1, feat0)


def _tc_layerout(p0, p1, d0, d1, feat0, h):
    BN = 2528

    def body(p0_r, p1_r, d0_r, d1_r, f0_r, h_r, o_r):
        agg = (p0_r[...] + p1_r[...]) / (d0_r[...] + d1_r[...] + 1e-9)
        f = ALPHA * f0_r[...] + (1.0 - ALPHA) * agg
        o_r[...] = jnp.where(f > 0, f, jnp.exp(f) - 1.0) + h_r[...]

    bs = pl.BlockSpec((BN, D), lambda i: (i, 0))
    return pl.pallas_call(
        body, grid=(NP // BN,), in_specs=[bs] * 6, out_specs=bs,
        out_shape=jax.ShapeDtypeStruct((NP, D), jnp.float32),
    )(p0, p1, d0, d1, feat0, h)


def kernel(ent_ids, rel_ids, edge_index, batch_cls, node_emb, rel_emb, params):
    S = jnp.asarray(_S128_np)
    src = edge_index[0]
    dst = edge_index[1]
    dst2 = jnp.reshape(dst, (E // 128, 128))
    ent_p = jnp.concatenate(
        [ent_ids, jnp.zeros((NG - N,), ent_ids.dtype)])

    h = _sc_gather(node_emb, ent_p, 80)[:NP]
    R = _tc_relproj(rel_emb, params['l0_Wr'])
    rel2 = jnp.reshape(
        _tc_rel2(jnp.reshape(rel_ids, (E // 128, 128))), (E,))
    re_full = _sc_gather_big(_tc_tile_r(R), rel2)

    for l in range(3):
        p = params
        q, k, v = _tc_qkv(h, p[f'l{l}_Wt'], p[f'l{l}_Wh'], p[f'l{l}_Wv'])
        ke = _sc_gather_big(k, src)
        qe = _sc_gather_big(q, dst)
        ex = _tc_sddmm(ke, qe, re_full if l == 0 else None, S)
        dpart = _sc_scatter_add(ex, dst2)
        d0, d1 = dpart[:NP], dpart[NP:]
        feat = v
        for hop in range(HOP):
            apart = _sc_hop_fused(feat, ex, src, dst2)
            if hop < HOP - 1:
                feat = _tc_combine(apart[:NP], apart[NP:], d0, d1, v)
            else:
                h = _tc_layerout(apart[:NP], apart[NP:], d0, d1, v, h)

    return _sc_gather(h, batch_cls, 8)
